# baseline (speedup 1.0000x reference)
import functools

import jax
import jax.numpy as jnp
from jax import lax
from jax.experimental import pallas as pl
from jax.experimental.pallas import tpu as pltpu
from jax.experimental.pallas import tpu_sc as plsc

N_ATOMS = 1_600_000
N_SYS = 100_000
NC = 2
NS = 16
NW = NC * NS
APW = N_ATOMS // NW
B = 10_000
NBLK = APW // B
L = 16

STRIPE = 6_240
TAIL_OFF = NS * STRIPE
TAIL = N_SYS - TAIL_OFF
RCHUNK = 3_120

_PARAMS = pltpu.CompilerParams(
    needs_layout_passes=False, use_tc_tiling_on_sc=False)


def _sc_partials(charge, pos_t, idx, zblock):
  mesh = plsc.VectorSubcoreMesh(core_axis_name="c", subcore_axis_name="s")

  buf = lambda dt=jnp.float32: pltpu.VMEM((B,), dt)

  @functools.partial(
      pl.kernel,
      out_type=jax.ShapeDtypeStruct((NC * 4 * N_SYS,), jnp.float32),
      mesh=mesh,
      compiler_params=_PARAMS,
      scratch_types=[
          pltpu.VMEM_SHARED((N_SYS,), jnp.float32),
          pltpu.VMEM_SHARED((N_SYS,), jnp.float32),
          pltpu.VMEM_SHARED((N_SYS,), jnp.float32),
          pltpu.VMEM_SHARED((N_SYS,), jnp.float32),
          buf(), buf(), buf(), buf(), buf(jnp.int32),
          buf(), buf(), buf(), buf(), buf(jnp.int32),
          pltpu.SemaphoreType.DMA,
          pltpu.SemaphoreType.DMA,
          pltpu.SemaphoreType.DMA,
          pltpu.SemaphoreType.DMA,
      ],
  )
  def k(q_hbm, pt_hbm, idx_hbm, z_hbm, out_hbm,
        acc0, acc1, acc2, acc3,
        q0, x0, y0, z0, i0, q1, x1, y1, z1, i1,
        sin0, sin1, ssc0, ssc1):
    cid = lax.axis_index("c")
    sid = lax.axis_index("s")
    wid = cid * NS + sid
    accs = (acc0, acc1, acc2, acc3)
    sets = ((q0, x0, y0, z0, i0), (q1, x1, y1, z1, i1))
    sem_in = (sin0, sin1)
    sem_sc = (ssc0, ssc1)

    pltpu.sync_copy(z_hbm, q0.at[pl.ds(0, STRIPE)])
    for acc in accs:
      pltpu.sync_copy(q0.at[pl.ds(0, STRIPE)],
                      acc.at[pl.ds(sid * STRIPE, STRIPE)])

    @pl.when(sid == 0)
    def _zero_tail():
      for acc in accs:
        pltpu.sync_copy(q0.at[pl.ds(0, TAIL)],
                        acc.at[pl.ds(TAIL_OFF, TAIL)])

    plsc.subcore_barrier()

    def fire_in(j):
      s = j % 2
      qb, xb, yb, zb, ib = sets[s]
      base = pl.multiple_of(wid * APW + j * B, 8)
      return [
          pltpu.async_copy(q_hbm.at[pl.ds(base, B)], qb, sem_in[s]),
          pltpu.async_copy(pt_hbm.at[0, pl.ds(base, B)], xb, sem_in[s]),
          pltpu.async_copy(pt_hbm.at[1, pl.ds(base, B)], yb, sem_in[s]),
          pltpu.async_copy(pt_hbm.at[2, pl.ds(base, B)], zb, sem_in[s]),
          pltpu.async_copy(idx_hbm.at[pl.ds(base, B)], ib, sem_in[s]),
      ]

    pending_in = [None, None]
    pending_sc = [None, None]
    pending_in[0] = fire_in(0)

    for j in range(NBLK):
      s = j % 2
      s2 = (j + 1) % 2
      qb, xb, yb, zb, ib = sets[s]

      for d in pending_in[s]:
        d.wait()
      pending_in[s] = None

      if pending_sc[s2] is not None:
        for d in pending_sc[s2]:
          d.wait()
        pending_sc[s2] = None
      if j + 1 < NBLK:
        pending_in[s2] = fire_in(j + 1)

      def step(i, carry):
        r0 = pl.multiple_of(i * L, L)
        q = qb[pl.ds(r0, L)]
        xb[pl.ds(r0, L)] = q * xb[pl.ds(r0, L)]
        yb[pl.ds(r0, L)] = q * yb[pl.ds(r0, L)]
        zb[pl.ds(r0, L)] = q * zb[pl.ds(r0, L)]
        return carry

      lax.fori_loop(0, B // L, step, 0)

      pending_sc[s] = [
          pltpu.async_copy(qb, acc0.at[ib], sem_sc[s], add=True),
          pltpu.async_copy(xb, acc1.at[ib], sem_sc[s], add=True),
          pltpu.async_copy(yb, acc2.at[ib], sem_sc[s], add=True),
          pltpu.async_copy(zb, acc3.at[ib], sem_sc[s], add=True),
      ]

    for s in range(2):
      if pending_sc[s] is not None:
        for d in pending_sc[s]:
          d.wait()

    plsc.subcore_barrier()

    bufs = (q0, x0, y0, z0)

    def writeout(row_off, n_rows):
      for c in range(4):
        pltpu.sync_copy(accs[c].at[pl.ds(row_off, n_rows)],
                        bufs[c].at[pl.ds(0, n_rows)])
        dst = pl.multiple_of(cid * 4 * N_SYS + c * N_SYS + row_off, 8)
        pltpu.sync_copy(bufs[c].at[pl.ds(0, n_rows)],
                        out_hbm.at[pl.ds(dst, n_rows)])

    writeout(sid * STRIPE, STRIPE)

    @pl.when(sid == 0)
    def _tail():
      writeout(TAIL_OFF, TAIL)

  return k(charge, pos_t, idx, zblock)


def _sc_merge(partials):
  mesh = plsc.VectorSubcoreMesh(core_axis_name="c", subcore_axis_name="s")

  @functools.partial(
      pl.kernel,
      out_type=jax.ShapeDtypeStruct((4 * N_SYS,), jnp.float32),
      mesh=mesh,
      compiler_params=_PARAMS,
      scratch_types=[
          pltpu.VMEM((RCHUNK,), jnp.float32),
          pltpu.VMEM((RCHUNK,), jnp.float32),
          pltpu.VMEM((RCHUNK,), jnp.float32),
      ],
  )
  def k(p_hbm, out_hbm, a_v, b_v, w_v):
    cid = lax.axis_index("c")
    sid = lax.axis_index("s")
    wid = cid * NS + sid

    def merge_rows(row_off, n_rows):
      for c in range(4):
        src_a = pl.multiple_of(c * N_SYS + row_off, 8)
        src_b = pl.multiple_of(4 * N_SYS + c * N_SYS + row_off, 8)
        pltpu.sync_copy(p_hbm.at[pl.ds(src_a, n_rows)],
                        a_v.at[pl.ds(0, n_rows)])
        pltpu.sync_copy(p_hbm.at[pl.ds(src_b, n_rows)],
                        b_v.at[pl.ds(0, n_rows)])

        def add_step(i, carry):
          r0 = pl.multiple_of(i * L, L)
          w_v[pl.ds(r0, L)] = a_v[pl.ds(r0, L)] + b_v[pl.ds(r0, L)]
          return carry

        lax.fori_loop(0, n_rows // L, add_step, 0)
        pltpu.sync_copy(w_v.at[pl.ds(0, n_rows)],
                        out_hbm.at[pl.ds(src_a, n_rows)])

    merge_rows(wid * RCHUNK, RCHUNK)

    @pl.when(wid == 0)
    def _tail():
      merge_rows(NW * RCHUNK, TAIL)

  return k(partials)


def kernel(per_atom_charge, positions, atomic_subsystem_indices,
           per_system_energy):
  idx = atomic_subsystem_indices.astype(jnp.int32)
  zblock = jnp.zeros((STRIPE,), jnp.float32)
  partials = _sc_partials(per_atom_charge, positions.T, idx, zblock)
  flat = _sc_merge(partials)
  return flat.reshape(4, N_SYS).T


# speedup vs baseline: 2.1899x; 2.1899x over previous
import functools

import jax
import jax.numpy as jnp
from jax import lax
from jax.experimental import pallas as pl
from jax.experimental.pallas import tpu as pltpu
from jax.experimental.pallas import tpu_sc as plsc

N_ATOMS = 1_600_000
N_SYS = 100_000
NC = 2
NS = 16
NW = NC * NS
APW = N_ATOMS // NW
B = 10_000
NBLK = APW // B
L = 16

STRIPE = 6_240
TAIL_OFF = NS * STRIPE
TAIL = N_SYS - TAIL_OFF
RCHUNK = 3_120

_PARAMS = pltpu.CompilerParams(
    needs_layout_passes=False, use_tc_tiling_on_sc=False)


def _sc_q_partials(charge, idx, zblock):
  mesh = plsc.VectorSubcoreMesh(core_axis_name="c", subcore_axis_name="s")

  @functools.partial(
      pl.kernel,
      out_type=jax.ShapeDtypeStruct((NC * N_SYS,), jnp.float32),
      mesh=mesh,
      compiler_params=_PARAMS,
      scratch_types=[
          pltpu.VMEM_SHARED((N_SYS,), jnp.float32),
          pltpu.VMEM((B,), jnp.float32),
          pltpu.VMEM((B,), jnp.int32),
          pltpu.VMEM((B,), jnp.float32),
          pltpu.VMEM((B,), jnp.int32),
          pltpu.SemaphoreType.DMA,
          pltpu.SemaphoreType.DMA,
          pltpu.SemaphoreType.DMA,
          pltpu.SemaphoreType.DMA,
      ],
  )
  def k(q_hbm, idx_hbm, z_hbm, out_hbm, acc, q0, i0, q1, i1,
        sin0, sin1, ssc0, ssc1):
    cid = lax.axis_index("c")
    sid = lax.axis_index("s")
    wid = cid * NS + sid
    sets = ((q0, i0), (q1, i1))
    sem_in = (sin0, sin1)
    sem_sc = (ssc0, ssc1)

    pltpu.sync_copy(z_hbm, q0.at[pl.ds(0, STRIPE)])
    pltpu.sync_copy(q0.at[pl.ds(0, STRIPE)],
                    acc.at[pl.ds(sid * STRIPE, STRIPE)])

    @pl.when(sid == 0)
    def _zero_tail():
      pltpu.sync_copy(q0.at[pl.ds(0, TAIL)], acc.at[pl.ds(TAIL_OFF, TAIL)])

    plsc.subcore_barrier()

    def fire_in(j):
      s = j % 2
      qb, ib = sets[s]
      base = pl.multiple_of(wid * APW + j * B, 8)
      return [
          pltpu.async_copy(q_hbm.at[pl.ds(base, B)], qb, sem_in[s]),
          pltpu.async_copy(idx_hbm.at[pl.ds(base, B)], ib, sem_in[s]),
      ]

    pending_in = [None, None]
    pending_sc = [None, None]
    pending_in[0] = fire_in(0)
    for j in range(NBLK):
      s = j % 2
      s2 = (j + 1) % 2
      qb, ib = sets[s]
      for d in pending_in[s]:
        d.wait()
      pending_in[s] = None
      if pending_sc[s2] is not None:
        for d in pending_sc[s2]:
          d.wait()
        pending_sc[s2] = None
      if j + 1 < NBLK:
        pending_in[s2] = fire_in(j + 1)
      pending_sc[s] = [pltpu.async_copy(qb, acc.at[ib], sem_sc[s], add=True)]
    for s in range(2):
      if pending_sc[s] is not None:
        for d in pending_sc[s]:
          d.wait()

    plsc.subcore_barrier()

    def writeout(row_off, n_rows):
      pltpu.sync_copy(acc.at[pl.ds(row_off, n_rows)], q0.at[pl.ds(0, n_rows)])
      dst = pl.multiple_of(cid * N_SYS + row_off, 8)
      pltpu.sync_copy(q0.at[pl.ds(0, n_rows)], out_hbm.at[pl.ds(dst, n_rows)])

    writeout(sid * STRIPE, STRIPE)

    @pl.when(sid == 0)
    def _tail():
      writeout(TAIL_OFF, TAIL)

  return k(charge, idx, zblock)


def _sc_partials(charge, px, py, pz, idx, zblock):
  mesh = plsc.VectorSubcoreMesh(core_axis_name="c", subcore_axis_name="s")

  buf = lambda dt=jnp.float32: pltpu.VMEM((B,), dt)

  @functools.partial(
      pl.kernel,
      out_type=jax.ShapeDtypeStruct((NC * 3 * N_SYS,), jnp.float32),
      mesh=mesh,
      compiler_params=_PARAMS,
      scratch_types=[
          pltpu.VMEM_SHARED((N_SYS,), jnp.float32),
          pltpu.VMEM_SHARED((N_SYS,), jnp.float32),
          pltpu.VMEM_SHARED((N_SYS,), jnp.float32),
          buf(), buf(), buf(), buf(), buf(jnp.int32),
          buf(), buf(), buf(), buf(), buf(jnp.int32),
          pltpu.SemaphoreType.DMA,
          pltpu.SemaphoreType.DMA,
          pltpu.SemaphoreType.DMA,
          pltpu.SemaphoreType.DMA,
      ],
  )
  def k(q_hbm, px_hbm, py_hbm, pz_hbm, idx_hbm, z_hbm, out_hbm,
        acc1, acc2, acc3,
        q0, x0, y0, z0, i0, q1, x1, y1, z1, i1,
        sin0, sin1, ssc0, ssc1):
    cid = lax.axis_index("c")
    sid = lax.axis_index("s")
    wid = cid * NS + sid
    accs = (acc1, acc2, acc3)
    sets = ((q0, x0, y0, z0, i0), (q1, x1, y1, z1, i1))
    sem_in = (sin0, sin1)
    sem_sc = (ssc0, ssc1)

    pltpu.sync_copy(z_hbm, q0.at[pl.ds(0, STRIPE)])
    for acc in accs:
      pltpu.sync_copy(q0.at[pl.ds(0, STRIPE)],
                      acc.at[pl.ds(sid * STRIPE, STRIPE)])

    @pl.when(sid == 0)
    def _zero_tail():
      for acc in accs:
        pltpu.sync_copy(q0.at[pl.ds(0, TAIL)],
                        acc.at[pl.ds(TAIL_OFF, TAIL)])

    plsc.subcore_barrier()

    def fire_in(j):
      s = j % 2
      qb, xb, yb, zb, ib = sets[s]
      base = pl.multiple_of(wid * APW + j * B, 8)
      return [
          pltpu.async_copy(q_hbm.at[pl.ds(base, B)], qb, sem_in[s]),
          pltpu.async_copy(px_hbm.at[pl.ds(base, B)], xb, sem_in[s]),
          pltpu.async_copy(py_hbm.at[pl.ds(base, B)], yb, sem_in[s]),
          pltpu.async_copy(pz_hbm.at[pl.ds(base, B)], zb, sem_in[s]),
          pltpu.async_copy(idx_hbm.at[pl.ds(base, B)], ib, sem_in[s]),
      ]

    pending_in = [None, None]
    pending_sc = [None, None]
    pending_in[0] = fire_in(0)

    for j in range(NBLK):
      s = j % 2
      s2 = (j + 1) % 2
      qb, xb, yb, zb, ib = sets[s]

      for d in pending_in[s]:
        d.wait()
      pending_in[s] = None

      if pending_sc[s2] is not None:
        for d in pending_sc[s2]:
          d.wait()
        pending_sc[s2] = None
      if j + 1 < NBLK:
        pending_in[s2] = fire_in(j + 1)

      def step(i, carry):
        r0 = pl.multiple_of(i * L, L)
        q = qb[pl.ds(r0, L)]
        xb[pl.ds(r0, L)] = q * xb[pl.ds(r0, L)]
        yb[pl.ds(r0, L)] = q * yb[pl.ds(r0, L)]
        zb[pl.ds(r0, L)] = q * zb[pl.ds(r0, L)]
        return carry

      lax.fori_loop(0, B // L, step, 0)

      pending_sc[s] = [
          pltpu.async_copy(xb, acc1.at[ib], sem_sc[s], add=True),
          pltpu.async_copy(yb, acc2.at[ib], sem_sc[s], add=True),
          pltpu.async_copy(zb, acc3.at[ib], sem_sc[s], add=True),
      ]

    for s in range(2):
      if pending_sc[s] is not None:
        for d in pending_sc[s]:
          d.wait()

    plsc.subcore_barrier()

    bufs = (x0, y0, z0)

    def writeout(row_off, n_rows):
      for c in range(3):
        pltpu.sync_copy(accs[c].at[pl.ds(row_off, n_rows)],
                        bufs[c].at[pl.ds(0, n_rows)])
        dst = pl.multiple_of(cid * 3 * N_SYS + c * N_SYS + row_off, 8)
        pltpu.sync_copy(bufs[c].at[pl.ds(0, n_rows)],
                        out_hbm.at[pl.ds(dst, n_rows)])

    writeout(sid * STRIPE, STRIPE)

    @pl.when(sid == 0)
    def _tail():
      writeout(TAIL_OFF, TAIL)

  return k(charge, px, py, pz, idx, zblock)


def _sc_merge(pq, pxyz):
  mesh = plsc.VectorSubcoreMesh(core_axis_name="c", subcore_axis_name="s")

  @functools.partial(
      pl.kernel,
      out_type=jax.ShapeDtypeStruct((4 * N_SYS,), jnp.float32),
      mesh=mesh,
      compiler_params=_PARAMS,
      scratch_types=[
          pltpu.VMEM((RCHUNK,), jnp.float32),
          pltpu.VMEM((RCHUNK,), jnp.float32),
          pltpu.VMEM((RCHUNK,), jnp.float32),
      ],
  )
  def k(pq_hbm, pxyz_hbm, out_hbm, a_v, b_v, w_v):
    cid = lax.axis_index("c")
    sid = lax.axis_index("s")
    wid = cid * NS + sid

    def merge_rows(row_off, n_rows):
      for c in range(4):
        if c == 0:
          src_ref = pq_hbm
          src_a = pl.multiple_of(row_off, 8)
          src_b = pl.multiple_of(N_SYS + row_off, 8)
        else:
          src_ref = pxyz_hbm
          src_a = pl.multiple_of((c - 1) * N_SYS + row_off, 8)
          src_b = pl.multiple_of(3 * N_SYS + (c - 1) * N_SYS + row_off, 8)
        pltpu.sync_copy(src_ref.at[pl.ds(src_a, n_rows)],
                        a_v.at[pl.ds(0, n_rows)])
        pltpu.sync_copy(src_ref.at[pl.ds(src_b, n_rows)],
                        b_v.at[pl.ds(0, n_rows)])

        def add_step(i, carry):
          r0 = pl.multiple_of(i * L, L)
          w_v[pl.ds(r0, L)] = a_v[pl.ds(r0, L)] + b_v[pl.ds(r0, L)]
          return carry

        lax.fori_loop(0, n_rows // L, add_step, 0)
        pltpu.sync_copy(w_v.at[pl.ds(0, n_rows)],
                        out_hbm.at[pl.ds(pl.multiple_of(c * N_SYS + row_off, 8),
                                         n_rows)])

    merge_rows(wid * RCHUNK, RCHUNK)

    @pl.when(wid == 0)
    def _tail():
      merge_rows(NW * RCHUNK, TAIL)

  return k(pq, pxyz)


def kernel(per_atom_charge, positions, atomic_subsystem_indices,
           per_system_energy):
  idx = atomic_subsystem_indices.astype(jnp.int32)
  zblock = jnp.zeros((STRIPE,), jnp.float32)
  px = positions[:, 0]
  py = positions[:, 1]
  pz = positions[:, 2]
  pq = _sc_q_partials(per_atom_charge, idx, zblock)
  pxyz = _sc_partials(per_atom_charge, px, py, pz, idx, zblock)
  flat = _sc_merge(pq, pxyz)
  return flat.reshape(4, N_SYS).T


# speedup vs baseline: 2.2759x; 1.0393x over previous
import functools

import jax
import jax.numpy as jnp
from jax import lax
from jax.experimental import pallas as pl
from jax.experimental.pallas import tpu as pltpu
from jax.experimental.pallas import tpu_sc as plsc

N_ATOMS = 1_600_000
N_SYS = 100_000
NC = 2
NS = 16
NW = NC * NS
APW = N_ATOMS // NW
B = 10_000
NBLK = APW // B
L = 16

STRIPE = 6_240
TAIL_OFF = NS * STRIPE
TAIL = N_SYS - TAIL_OFF
RCHUNK = 3_120

_PARAMS = pltpu.CompilerParams(
    needs_layout_passes=False, use_tc_tiling_on_sc=False)


def _sc_partials(charge, px, py, pz, idx, zblock):
  mesh = plsc.VectorSubcoreMesh(core_axis_name="c", subcore_axis_name="s")

  buf = lambda dt=jnp.float32: pltpu.VMEM((B,), dt)

  @functools.partial(
      pl.kernel,
      out_type=jax.ShapeDtypeStruct((NC * 4 * N_SYS,), jnp.float32),
      mesh=mesh,
      compiler_params=_PARAMS,
      scratch_types=[
          pltpu.VMEM_SHARED((N_SYS,), jnp.float32),
          pltpu.VMEM_SHARED((N_SYS,), jnp.float32),
          pltpu.VMEM_SHARED((N_SYS,), jnp.float32),
          pltpu.VMEM_SHARED((N_SYS,), jnp.float32),
          buf(), buf(), buf(), buf(), buf(jnp.int32),
          buf(), buf(), buf(), buf(), buf(jnp.int32),
          pltpu.SemaphoreType.DMA,
          pltpu.SemaphoreType.DMA,
          pltpu.SemaphoreType.DMA,
          pltpu.SemaphoreType.DMA,
      ],
  )
  def k(q_hbm, px_hbm, py_hbm, pz_hbm, idx_hbm, z_hbm, out_hbm,
        acc0, acc1, acc2, acc3,
        q0, x0, y0, z0, i0, q1, x1, y1, z1, i1,
        sin0, sin1, ssc0, ssc1):
    cid = lax.axis_index("c")
    sid = lax.axis_index("s")
    wid = cid * NS + sid
    accs = (acc0, acc1, acc2, acc3)
    sets = ((q0, x0, y0, z0, i0), (q1, x1, y1, z1, i1))
    sem_in = (sin0, sin1)
    sem_sc = (ssc0, ssc1)

    pltpu.sync_copy(z_hbm, q0.at[pl.ds(0, STRIPE)])
    for acc in accs:
      pltpu.sync_copy(q0.at[pl.ds(0, STRIPE)],
                      acc.at[pl.ds(sid * STRIPE, STRIPE)])

    @pl.when(sid == 0)
    def _zero_tail():
      for acc in accs:
        pltpu.sync_copy(q0.at[pl.ds(0, TAIL)],
                        acc.at[pl.ds(TAIL_OFF, TAIL)])

    plsc.subcore_barrier()

    def fire_in(j):
      s = j % 2
      qb, xb, yb, zb, ib = sets[s]
      base = pl.multiple_of(wid * APW + j * B, 8)
      return [
          pltpu.async_copy(q_hbm.at[pl.ds(base, B)], qb, sem_in[s]),
          pltpu.async_copy(px_hbm.at[pl.ds(base, B)], xb, sem_in[s]),
          pltpu.async_copy(py_hbm.at[pl.ds(base, B)], yb, sem_in[s]),
          pltpu.async_copy(pz_hbm.at[pl.ds(base, B)], zb, sem_in[s]),
          pltpu.async_copy(idx_hbm.at[pl.ds(base, B)], ib, sem_in[s]),
      ]

    pending_in = [None, None]
    pending_sc = [None, None]
    pending_in[0] = fire_in(0)

    for j in range(NBLK):
      s = j % 2
      s2 = (j + 1) % 2
      qb, xb, yb, zb, ib = sets[s]

      for d in pending_in[s]:
        d.wait()
      pending_in[s] = None

      if pending_sc[s2] is not None:
        for d in pending_sc[s2]:
          d.wait()
        pending_sc[s2] = None
      if j + 1 < NBLK:
        pending_in[s2] = fire_in(j + 1)

      def step(i, carry):
        r0 = pl.multiple_of(i * L, L)
        q = qb[pl.ds(r0, L)]
        xb[pl.ds(r0, L)] = q * xb[pl.ds(r0, L)]
        yb[pl.ds(r0, L)] = q * yb[pl.ds(r0, L)]
        zb[pl.ds(r0, L)] = q * zb[pl.ds(r0, L)]
        return carry

      lax.fori_loop(0, B // L, step, 0)

      pending_sc[s] = [
          pltpu.async_copy(qb, acc0.at[ib], sem_sc[s], add=True),
          pltpu.async_copy(xb, acc1.at[ib], sem_sc[s], add=True),
          pltpu.async_copy(yb, acc2.at[ib], sem_sc[s], add=True),
          pltpu.async_copy(zb, acc3.at[ib], sem_sc[s], add=True),
      ]

    for s in range(2):
      if pending_sc[s] is not None:
        for d in pending_sc[s]:
          d.wait()

    plsc.subcore_barrier()

    bufs = (q0, x0, y0, z0)

    def writeout(row_off, n_rows):
      for c in range(4):
        pltpu.sync_copy(accs[c].at[pl.ds(row_off, n_rows)],
                        bufs[c].at[pl.ds(0, n_rows)])
        dst = pl.multiple_of(cid * 4 * N_SYS + c * N_SYS + row_off, 8)
        pltpu.sync_copy(bufs[c].at[pl.ds(0, n_rows)],
                        out_hbm.at[pl.ds(dst, n_rows)])

    writeout(sid * STRIPE, STRIPE)

    @pl.when(sid == 0)
    def _tail():
      writeout(TAIL_OFF, TAIL)

  return k(charge, px, py, pz, idx, zblock)


def _sc_merge(partials):
  mesh = plsc.VectorSubcoreMesh(core_axis_name="c", subcore_axis_name="s")

  @functools.partial(
      pl.kernel,
      out_type=jax.ShapeDtypeStruct((4 * N_SYS,), jnp.float32),
      mesh=mesh,
      compiler_params=_PARAMS,
      scratch_types=[
          pltpu.VMEM((RCHUNK,), jnp.float32),
          pltpu.VMEM((RCHUNK,), jnp.float32),
          pltpu.VMEM((RCHUNK,), jnp.float32),
      ],
  )
  def k(p_hbm, out_hbm, a_v, b_v, w_v):
    cid = lax.axis_index("c")
    sid = lax.axis_index("s")
    wid = cid * NS + sid

    def merge_rows(row_off, n_rows):
      for c in range(4):
        src_a = pl.multiple_of(c * N_SYS + row_off, 8)
        src_b = pl.multiple_of(4 * N_SYS + c * N_SYS + row_off, 8)
        pltpu.sync_copy(p_hbm.at[pl.ds(src_a, n_rows)],
                        a_v.at[pl.ds(0, n_rows)])
        pltpu.sync_copy(p_hbm.at[pl.ds(src_b, n_rows)],
                        b_v.at[pl.ds(0, n_rows)])

        def add_step(i, carry):
          r0 = pl.multiple_of(i * L, L)
          w_v[pl.ds(r0, L)] = a_v[pl.ds(r0, L)] + b_v[pl.ds(r0, L)]
          return carry

        lax.fori_loop(0, n_rows // L, add_step, 0)
        pltpu.sync_copy(w_v.at[pl.ds(0, n_rows)],
                        out_hbm.at[pl.ds(src_a, n_rows)])

    merge_rows(wid * RCHUNK, RCHUNK)

    @pl.when(wid == 0)
    def _tail():
      merge_rows(NW * RCHUNK, TAIL)

  return k(partials)


def kernel(per_atom_charge, positions, atomic_subsystem_indices,
           per_system_energy):
  idx = atomic_subsystem_indices.astype(jnp.int32)
  zblock = jnp.zeros((STRIPE,), jnp.float32)
  px = positions[:, 0]
  py = positions[:, 1]
  pz = positions[:, 2]
  partials = _sc_partials(per_atom_charge, px, py, pz, idx, zblock)
  flat = _sc_merge(partials)
  return flat.reshape(4, N_SYS).T
